# baseline (speedup 1.0000x reference)
import functools

import jax
import jax.numpy as jnp
from jax import lax
from jax.experimental import pallas as pl
from jax.experimental.pallas import tpu as pltpu
from jax.experimental.pallas import tpu_sc as plsc

N = 10000
E = 320000
VEC_IN = 16
SCALAR_IN = 128
EDGE_IN = 16
HID = 64
NH = 4
T = 5
SCALAR_OUT = 128
ROW = 128

NC = 2
NS = 16
NW = NC * NS
CH = 80
NSLICE = 5
ES = E // NSLICE
EWS = ES // NW
NCHS = EWS // CH
NPS = 632
N_PAD = NPS * NS


def _silu(x):
    return x * (1.0 / (1.0 + jnp.exp(-x)))


def _pre_body(h_ref, zf_ref, w1a_ref, w1b_ref, b1_ref, ts_ref, td_ref):
    bn = h_ref.shape[0]
    h = h_ref[...]
    zf = zf_ref[...]
    pad = jnp.zeros((bn, 16), jnp.float32)
    hb = jnp.dot(h, w1b_ref[...], preferred_element_type=jnp.float32)
    ha = jnp.dot(h, w1a_ref[...], preferred_element_type=jnp.float32) + b1_ref[...]
    ts_ref[...] = jnp.concatenate([hb, zf, pad], axis=1)
    td_ref[...] = jnp.concatenate([ha, zf, pad], axis=1)


def _tc_pre(h, zf, w1a, w1b, b1, *, bn=2000, interpret=False):
    row_spec = lambda w: pl.BlockSpec((bn, w), lambda i: (i, 0))
    full = lambda a: pl.BlockSpec(a.shape, lambda i: (0,) * a.ndim)
    return pl.pallas_call(
        _pre_body,
        grid=(N // bn,),
        in_specs=[row_spec(SCALAR_IN), row_spec(48), full(w1a), full(w1b),
                  full(b1)],
        out_specs=[row_spec(SCALAR_IN), row_spec(SCALAR_IN)],
        out_shape=[jax.ShapeDtypeStruct((N, SCALAR_IN), jnp.float32),
                   jax.ShapeDtypeStruct((N, SCALAR_IN), jnp.float32)],
        interpret=interpret,
    )(h, zf, w1a, w1b, b1)


def _sc_gather(ts, td, src, dst):
    mesh = plsc.VectorSubcoreMesh(core_axis_name="c", subcore_axis_name="s")

    @functools.partial(
        pl.kernel,
        out_type=(
            jax.ShapeDtypeStruct((ES, SCALAR_IN), jnp.float32),
            jax.ShapeDtypeStruct((ES, SCALAR_IN), jnp.float32),
        ),
        mesh=mesh,
        scratch_types=[
            pltpu.VMEM((EWS,), jnp.int32),
            pltpu.VMEM((EWS,), jnp.int32),
            pltpu.VMEM((CH, SCALAR_IN), jnp.float32),
            pltpu.VMEM((CH, SCALAR_IN), jnp.float32),
            pltpu.VMEM((CH, SCALAR_IN), jnp.float32),
            pltpu.VMEM((CH, SCALAR_IN), jnp.float32),
            pltpu.SemaphoreType.DMA,
            pltpu.SemaphoreType.DMA,
            pltpu.SemaphoreType.DMA,
            pltpu.SemaphoreType.DMA,
        ],
    )
    def k(ts_hbm, td_hbm, src_hbm, dst_hbm, gs_hbm, gd_hbm,
          idx_s, idx_d, sb0, db0, sb1, db1, gsem0, gsem1, wsem0, wsem1):
        wid = lax.axis_index("s") * NC + lax.axis_index("c")
        wbase = wid * EWS
        cp_i = pltpu.async_copy(src_hbm.at[pl.ds(wbase, EWS)], idx_s, gsem0)
        cp_j = pltpu.async_copy(dst_hbm.at[pl.ds(wbase, EWS)], idx_d, gsem1)
        cp_i.wait()
        cp_j.wait()

        sets = ((sb0, db0, gsem0, wsem0), (sb1, db1, gsem1, wsem1))

        def issue_gather(ci):
            sb, db, gsem, _ = sets[ci % 2]
            gh1 = pltpu.async_copy(ts_hbm.at[idx_s.at[pl.ds(ci * CH, CH)]],
                                   sb, gsem)
            gh2 = pltpu.async_copy(td_hbm.at[idx_d.at[pl.ds(ci * CH, CH)]],
                                   db, gsem)
            return gh1, gh2

        gh = {0: issue_gather(0)}
        wh = {}
        for j in range(NCHS):
            sb, db, _, wsem = sets[j % 2]
            if j + 1 < NCHS:
                if j - 1 in wh:
                    for hnd in wh.pop(j - 1):
                        hnd.wait()
                gh[j + 1] = issue_gather(j + 1)
            for hnd in gh.pop(j):
                hnd.wait()
            base = wbase + j * CH
            wh[j] = (pltpu.async_copy(sb, gs_hbm.at[pl.ds(base, CH)], wsem),
                     pltpu.async_copy(db, gd_hbm.at[pl.ds(base, CH)], wsem))
        for j in wh:
            for hnd in wh[j]:
                hnd.wait()

    return k(ts, td, src, dst)


def _sel_mats():
    import numpy as np

    def zrow(d, t, h):
        return d * 16 + t * 4 + h if t < 4 else 48 + d

    Wp = np.zeros((51, 300), np.float32)
    Wq = np.zeros((51, 300), np.float32)
    Wr = np.zeros((300, 100), np.float32)
    for d in range(3):
        for t in range(T):
            for r in range(T):
                for h in range(NH):
                    c = d * 100 + (t * T + r) * NH + h
                    Wp[zrow(d, t, h), c] = 1.0
                    Wq[zrow(d, r, h), c] = 1.0
                    Wr[c, (t * T + r) * NH + h] = 1.0
    Wn = np.ones((100, 8), np.float32)
    Wa = np.zeros((51, 240), np.float32)
    Wb = np.zeros((80, 240), np.float32)
    Wz = np.zeros((240, 48), np.float32)
    for d in range(3):
        for t in range(T):
            for k in range(4):
                for h in range(NH):
                    c = d * 80 + t * 16 + k * 4 + h
                    Wa[zrow(d, t, h), c] = 1.0
                    Wb[t * 16 + k * 4 + h, c] = 1.0
                    Wz[c, d * 16 + k * 4 + h] = 1.0
    return (jnp.asarray(Wp, jnp.bfloat16), jnp.asarray(Wq, jnp.bfloat16),
            jnp.asarray(Wr), jnp.asarray(Wn),
            jnp.asarray(Wa, jnp.bfloat16), jnp.asarray(Wb, jnp.bfloat16),
            jnp.asarray(Wz))


def _edge_body(gs_ref, gd_ref, edf_ref, ev_ref,
               efW_ref, efb_ref, w1in_ref, w1ef_ref,
               w2_ref, b2_ref, vw1_ref, vb1_ref, vw2_ref, vb2_ref,
               wp_ref, wq_ref, wr_ref, wn_ref, wa_ref, wb_ref, wz_ref,
               out_ref):
    dot = lambda a, b: jnp.dot(a, b, preferred_element_type=jnp.float32)
    gs = gs_ref[...]
    gd = gd_ref[...]
    zdiff = gd[:, HID:HID + 48] - gs[:, HID:HID + 48]
    zcat = jnp.concatenate([zdiff, ev_ref[...]], axis=1)
    z16 = zcat.astype(jnp.bfloat16)
    P = dot(z16, wp_ref[...])
    Q = dot(z16, wq_ref[...])
    inv = dot(P * Q, wr_ref[...])
    n2 = dot(inv * inv, wn_ref[...])[:, :1]
    inv = inv / jnp.maximum(jnp.sqrt(n2), 1e-12)

    ef = jnp.dot(edf_ref[...], efW_ref[...],
                 preferred_element_type=jnp.float32) + efb_ref[...]
    pre = (gd[:, :HID] + gs[:, :HID]
           + jnp.dot(inv, w1in_ref[...], preferred_element_type=jnp.float32)
           + jnp.dot(ef, w1ef_ref[...], preferred_element_type=jnp.float32))
    msg = _silu(pre)
    msg = _silu(jnp.dot(msg, w2_ref[...], preferred_element_type=jnp.float32)
                + b2_ref[...])
    bas = jnp.dot(_silu(jnp.dot(msg, vw1_ref[...],
                                preferred_element_type=jnp.float32) + vb1_ref[...]),
                  vw2_ref[...], preferred_element_type=jnp.float32) + vb2_ref[...]
    A = dot(z16, wa_ref[...])
    B = dot(bas.astype(jnp.bfloat16), wb_ref[...])
    za = dot(A * B, wz_ref[...])
    ones = jnp.ones((gs.shape[0], 16), jnp.float32)
    out_ref[...] = jnp.concatenate([msg, za, ones], axis=1)


def _tc_edge(gs, gd, edf, ev, efW, efb, w1in, w1ef,
             w2, b2, vw1, vb1, vw2, vb2, *, be=1280, interpret=False):
    ne = gs.shape[0]
    grid = (ne // be,)
    sel = _sel_mats()
    row_spec = lambda w: pl.BlockSpec((be, w), lambda i: (i, 0))
    full = lambda a: pl.BlockSpec(a.shape, lambda i: (0,) * a.ndim)
    consts = (efW, efb, w1in, w1ef, w2, b2, vw1, vb1, vw2, vb2) + sel
    return pl.pallas_call(
        _edge_body,
        grid=grid,
        in_specs=[row_spec(SCALAR_IN), row_spec(SCALAR_IN),
                  row_spec(EDGE_IN), row_spec(3)] + [full(c) for c in consts],
        out_specs=row_spec(ROW),
        out_shape=jax.ShapeDtypeStruct((ne, ROW), jnp.float32),
        interpret=interpret,
    )(gs, gd, edf, ev, *consts)


def _sc_scatter(eo, dst):
    mesh = plsc.VectorSubcoreMesh(core_axis_name="c", subcore_axis_name="s")

    @functools.partial(
        pl.kernel,
        out_type=jax.ShapeDtypeStruct((NC, N_PAD, ROW), jnp.float32),
        mesh=mesh,
        scratch_types=[
            pltpu.VMEM((CH,), jnp.int32),
            pltpu.VMEM((CH,), jnp.int32),
            pltpu.VMEM((CH, ROW), jnp.float32),
            pltpu.VMEM((CH, ROW), jnp.float32),
            pltpu.VMEM((8, ROW), jnp.float32),
            pltpu.VMEM_SHARED((N_PAD, ROW), jnp.float32),
            pltpu.SemaphoreType.DMA,
            pltpu.SemaphoreType.DMA,
        ],
    )
    def k(eo_hbm, dst_hbm, part_hbm, idx0, idx1, rows0, rows1, zb, accum,
          lsem0, lsem1):
        cid = lax.axis_index("c")
        sid = lax.axis_index("s")
        wid = sid * NC + cid

        @pl.loop(0, 8)
        def _(i):
            for j in range(ROW // 16):
                zb[i, pl.ds(j * 16, 16)] = jnp.zeros((16,), jnp.float32)

        @pl.loop(0, NPS // 8)
        def _(t):
            pltpu.sync_copy(zb, accum.at[pl.ds(sid * NPS + t * 8, 8)])

        plsc.subcore_barrier()

        sets = ((idx0, rows0, lsem0), (idx1, rows1, lsem1))

        def issue_loads(ci):
            idx_v, rows, lsem = sets[ci % 2]
            base = wid * EWS + ci * CH
            lh1 = pltpu.async_copy(dst_hbm.at[pl.ds(base, CH)], idx_v, lsem)
            lh2 = pltpu.async_copy(eo_hbm.at[pl.ds(base, CH)], rows, lsem)
            return lh1, lh2

        lh = {0: issue_loads(0)}
        for j in range(NCHS):
            idx_v, rows, _ = sets[j % 2]
            if j + 1 < NCHS:
                lh[j + 1] = issue_loads(j + 1)
            for hnd in lh.pop(j):
                hnd.wait()
            pltpu.sync_copy(rows, accum.at[idx_v], add=True)

        plsc.subcore_barrier()
        pltpu.sync_copy(accum.at[pl.ds(sid * NPS, NPS)],
                        part_hbm.at[cid, pl.ds(sid * NPS, NPS)])

    return k(eo, dst)


def _node_body(*refs):
    part_refs = refs[:-8]
    h_ref, wh_ref, wm_ref, b1_ref, w2_ref, b2_ref, z_ref, h_out_ref = refs[-8:]
    acc = None
    for pr in part_refs:
        pall = pr[...]
        for c in range(NC):
            p = pall[c]
            acc = p if acc is None else acc + p
    m = acc[:, :HID]
    zsum = acc[:, HID:HID + 48]
    cnt = acc[:, HID + 48:HID + 49]
    z_ref[...] = zsum / jnp.maximum(cnt, 1.0)
    pre = (jnp.dot(h_ref[...], wh_ref[...], preferred_element_type=jnp.float32)
           + jnp.dot(m, wm_ref[...], preferred_element_type=jnp.float32)
           + b1_ref[...])
    h_out_ref[...] = (jnp.dot(_silu(pre), w2_ref[...],
                              preferred_element_type=jnp.float32) + b2_ref[...])


def _tc_node(parts, h, wh, wm, b1, w2, b2, *, bn=1000, interpret=False):
    grid = (N // bn,)
    row_spec = lambda w: pl.BlockSpec((bn, w), lambda i: (i, 0))
    part_spec = pl.BlockSpec((NC, bn, ROW), lambda i: (0, i, 0))
    full = lambda a: pl.BlockSpec(a.shape, lambda i: (0,) * a.ndim)
    return pl.pallas_call(
        _node_body,
        grid=grid,
        in_specs=[part_spec] * len(parts)
        + [row_spec(SCALAR_IN), full(wh), full(wm), full(b1), full(w2),
           full(b2)],
        out_specs=[row_spec(48), row_spec(SCALAR_IN)],
        out_shape=[jax.ShapeDtypeStruct((N, 48), jnp.float32),
                   jax.ShapeDtypeStruct((N, SCALAR_IN), jnp.float32)],
        interpret=interpret,
    )(*parts, h, wh, wm, b1, w2, b2)


def kernel(Z, h, edge_index, edge_distance_feature, edge_distance_vec,
           edge_distance, edge_fc_W, edge_fc_b, msg_W1, msg_b1, msg_W2, msg_b2,
           vec_W1, vec_b1, vec_W2, vec_b2, sc_W1, sc_b1, sc_W2, sc_b2):
    src = edge_index[0]
    dst = edge_index[1]
    zf = Z.reshape(N, 48)

    ts, td = _tc_pre(h, zf, msg_W1[0:128], msg_W1[128:256],
                     msg_b1.reshape(1, HID))

    parts = []
    for s in range(NSLICE):
        lo, hi = s * ES, (s + 1) * ES
        src_s, dst_s = src[lo:hi], dst[lo:hi]
        gs, gd = _sc_gather(ts, td, src_s, dst_s)
        eo = _tc_edge(
            gs, gd, edge_distance_feature[lo:hi], edge_distance_vec[lo:hi],
            edge_fc_W, edge_fc_b.reshape(1, HID),
            msg_W1[256:356], msg_W1[356:420],
            msg_W2, msg_b2.reshape(1, HID),
            vec_W1, vec_b1.reshape(1, HID), vec_W2, vec_b2.reshape(1, 80))
        parts.append(_sc_scatter(eo, dst_s))

    z_out, h_out = _tc_node(
        parts, h,
        sc_W1[0:SCALAR_IN], sc_W1[SCALAR_IN:SCALAR_IN + HID],
        sc_b1.reshape(1, HID), sc_W2, sc_b2.reshape(1, SCALAR_OUT))

    return (z_out.reshape(N, 3, VEC_IN), h_out)


# speedup vs baseline: 1.0293x; 1.0293x over previous
import functools

import jax
import jax.numpy as jnp
from jax import lax
from jax.experimental import pallas as pl
from jax.experimental.pallas import tpu as pltpu
from jax.experimental.pallas import tpu_sc as plsc

N = 10000
E = 320000
VEC_IN = 16
SCALAR_IN = 128
EDGE_IN = 16
HID = 64
NH = 4
T = 5
SCALAR_OUT = 128
ROW = 128

NC = 2
NS = 16
NW = NC * NS
CH = 80
NSLICE = 5
ES = E // NSLICE
EWS = ES // NW
NCHS = EWS // CH
NPS = 632
N_PAD = NPS * NS


def _silu(x):
    return x * (1.0 / (1.0 + jnp.exp(-x)))


def _pre_body(h_ref, zf_ref, w1a_ref, w1b_ref, b1_ref, ts_ref, td_ref):
    bn = h_ref.shape[0]
    h = h_ref[...]
    zf = zf_ref[...]
    pad = jnp.zeros((bn, 16), jnp.float32)
    hb = jnp.dot(h, w1b_ref[...], preferred_element_type=jnp.float32)
    ha = jnp.dot(h, w1a_ref[...], preferred_element_type=jnp.float32) + b1_ref[...]
    ts_ref[...] = jnp.concatenate([hb, zf, pad], axis=1)
    td_ref[...] = jnp.concatenate([ha, zf, pad], axis=1)


def _tc_pre(h, zf, w1a, w1b, b1, *, bn=2000, interpret=False):
    row_spec = lambda w: pl.BlockSpec((bn, w), lambda i: (i, 0))
    full = lambda a: pl.BlockSpec(a.shape, lambda i: (0,) * a.ndim)
    return pl.pallas_call(
        _pre_body,
        grid=(N // bn,),
        in_specs=[row_spec(SCALAR_IN), row_spec(48), full(w1a), full(w1b),
                  full(b1)],
        out_specs=[row_spec(SCALAR_IN), row_spec(SCALAR_IN)],
        out_shape=[jax.ShapeDtypeStruct((N, SCALAR_IN), jnp.float32),
                   jax.ShapeDtypeStruct((N, SCALAR_IN), jnp.float32)],
        interpret=interpret,
    )(h, zf, w1a, w1b, b1)


def _sc_gather(ts, td, src, dst):
    mesh = plsc.VectorSubcoreMesh(core_axis_name="c", subcore_axis_name="s")

    @functools.partial(
        pl.kernel,
        out_type=(
            jax.ShapeDtypeStruct((ES, SCALAR_IN), jnp.float32),
            jax.ShapeDtypeStruct((ES, SCALAR_IN), jnp.float32),
        ),
        mesh=mesh,
        scratch_types=[
            pltpu.VMEM((EWS,), jnp.int32),
            pltpu.VMEM((EWS,), jnp.int32),
            pltpu.VMEM((CH, SCALAR_IN), jnp.float32),
            pltpu.VMEM((CH, SCALAR_IN), jnp.float32),
            pltpu.SemaphoreType.DMA,
            pltpu.SemaphoreType.DMA,
        ],
    )
    def k(ts_hbm, td_hbm, src_hbm, dst_hbm, gs_hbm, gd_hbm,
          idx_s, idx_d, sb, db, sem1, sem2):
        wid = lax.axis_index("s") * NC + lax.axis_index("c")
        wbase = wid * EWS
        cp_i = pltpu.async_copy(src_hbm.at[pl.ds(wbase, EWS)], idx_s, sem1)
        cp_j = pltpu.async_copy(dst_hbm.at[pl.ds(wbase, EWS)], idx_d, sem2)
        cp_i.wait()
        cp_j.wait()

        @pl.loop(0, NCHS)
        def _(ci):
            base = wbase + ci * CH
            cp_s = pltpu.async_copy(ts_hbm.at[idx_s.at[pl.ds(ci * CH, CH)]],
                                    sb, sem1)
            cp_d = pltpu.async_copy(td_hbm.at[idx_d.at[pl.ds(ci * CH, CH)]],
                                    db, sem2)
            cp_s.wait()
            cp_d.wait()
            pltpu.sync_copy(sb, gs_hbm.at[pl.ds(base, CH)])
            pltpu.sync_copy(db, gd_hbm.at[pl.ds(base, CH)])

    return k(ts, td, src, dst)


def _sel_mats():
    import numpy as np

    def zrow(d, t, h):
        return d * 16 + t * 4 + h if t < 4 else 48 + d

    Wp = np.zeros((51, 300), np.float32)
    Wq = np.zeros((51, 300), np.float32)
    Wr = np.zeros((300, 100), np.float32)
    for d in range(3):
        for t in range(T):
            for r in range(T):
                for h in range(NH):
                    c = d * 100 + (t * T + r) * NH + h
                    Wp[zrow(d, t, h), c] = 1.0
                    Wq[zrow(d, r, h), c] = 1.0
                    Wr[c, (t * T + r) * NH + h] = 1.0
    Wn = np.ones((100, 8), np.float32)
    Wa = np.zeros((51, 240), np.float32)
    Wb = np.zeros((80, 240), np.float32)
    Wz = np.zeros((240, 48), np.float32)
    for d in range(3):
        for t in range(T):
            for k in range(4):
                for h in range(NH):
                    c = d * 80 + t * 16 + k * 4 + h
                    Wa[zrow(d, t, h), c] = 1.0
                    Wb[t * 16 + k * 4 + h, c] = 1.0
                    Wz[c, d * 16 + k * 4 + h] = 1.0
    return (jnp.asarray(Wp, jnp.bfloat16), jnp.asarray(Wq, jnp.bfloat16),
            jnp.asarray(Wr), jnp.asarray(Wn),
            jnp.asarray(Wa, jnp.bfloat16), jnp.asarray(Wb, jnp.bfloat16),
            jnp.asarray(Wz))


def _edge_body(gs_ref, gd_ref, edf_ref, ev_ref,
               efW_ref, efb_ref, w1in_ref, w1ef_ref,
               w2_ref, b2_ref, vw1_ref, vb1_ref, vw2_ref, vb2_ref,
               wp_ref, wq_ref, wr_ref, wn_ref, wa_ref, wb_ref, wz_ref,
               out_ref):
    dot = lambda a, b: jnp.dot(a, b, preferred_element_type=jnp.float32)
    gs = gs_ref[...]
    gd = gd_ref[...]
    zdiff = gd[:, HID:HID + 48] - gs[:, HID:HID + 48]
    zcat = jnp.concatenate([zdiff, ev_ref[...]], axis=1)
    z16 = zcat.astype(jnp.bfloat16)
    P = dot(z16, wp_ref[...])
    Q = dot(z16, wq_ref[...])
    inv = dot(P * Q, wr_ref[...])
    n2 = dot(inv * inv, wn_ref[...])[:, :1]
    inv = inv / jnp.maximum(jnp.sqrt(n2), 1e-12)

    ef = jnp.dot(edf_ref[...], efW_ref[...],
                 preferred_element_type=jnp.float32) + efb_ref[...]
    pre = (gd[:, :HID] + gs[:, :HID]
           + jnp.dot(inv, w1in_ref[...], preferred_element_type=jnp.float32)
           + jnp.dot(ef, w1ef_ref[...], preferred_element_type=jnp.float32))
    msg = _silu(pre)
    msg = _silu(jnp.dot(msg, w2_ref[...], preferred_element_type=jnp.float32)
                + b2_ref[...])
    bas = jnp.dot(_silu(jnp.dot(msg, vw1_ref[...],
                                preferred_element_type=jnp.float32) + vb1_ref[...]),
                  vw2_ref[...], preferred_element_type=jnp.float32) + vb2_ref[...]
    A = dot(z16, wa_ref[...])
    B = dot(bas.astype(jnp.bfloat16), wb_ref[...])
    za = dot(A * B, wz_ref[...])
    ones = jnp.ones((gs.shape[0], 16), jnp.float32)
    out_ref[...] = jnp.concatenate([msg, za, ones], axis=1)


def _tc_edge(gs, gd, edf, ev, efW, efb, w1in, w1ef,
             w2, b2, vw1, vb1, vw2, vb2, *, be=1280, interpret=False):
    ne = gs.shape[0]
    grid = (ne // be,)
    sel = _sel_mats()
    row_spec = lambda w: pl.BlockSpec((be, w), lambda i: (i, 0))
    full = lambda a: pl.BlockSpec(a.shape, lambda i: (0,) * a.ndim)
    consts = (efW, efb, w1in, w1ef, w2, b2, vw1, vb1, vw2, vb2) + sel
    return pl.pallas_call(
        _edge_body,
        grid=grid,
        in_specs=[row_spec(SCALAR_IN), row_spec(SCALAR_IN),
                  row_spec(EDGE_IN), row_spec(3)] + [full(c) for c in consts],
        out_specs=row_spec(ROW),
        out_shape=jax.ShapeDtypeStruct((ne, ROW), jnp.float32),
        interpret=interpret,
    )(gs, gd, edf, ev, *consts)


def _sc_scatter(eo, dst):
    mesh = plsc.VectorSubcoreMesh(core_axis_name="c", subcore_axis_name="s")

    @functools.partial(
        pl.kernel,
        out_type=jax.ShapeDtypeStruct((NC, N_PAD, ROW), jnp.float32),
        mesh=mesh,
        scratch_types=[
            pltpu.VMEM((CH,), jnp.int32),
            pltpu.VMEM((CH, ROW), jnp.float32),
            pltpu.VMEM((8, ROW), jnp.float32),
            pltpu.VMEM_SHARED((N_PAD, ROW), jnp.float32),
            pltpu.SemaphoreType.DMA,
        ],
    )
    def k(eo_hbm, dst_hbm, part_hbm, idx_v, rows, zb, accum, sem):
        cid = lax.axis_index("c")
        sid = lax.axis_index("s")
        wid = sid * NC + cid

        @pl.loop(0, 8)
        def _(i):
            for j in range(ROW // 16):
                zb[i, pl.ds(j * 16, 16)] = jnp.zeros((16,), jnp.float32)

        zh = [pltpu.async_copy(zb, accum.at[pl.ds(sid * NPS + t * 8, 8)], sem)
              for t in range(NPS // 8)]
        for hnd in zh:
            hnd.wait()

        plsc.subcore_barrier()

        @pl.loop(0, NCHS)
        def _(ci):
            base = wid * EWS + ci * CH
            pltpu.sync_copy(dst_hbm.at[pl.ds(base, CH)], idx_v)
            pltpu.sync_copy(eo_hbm.at[pl.ds(base, CH)], rows)
            pltpu.sync_copy(rows, accum.at[idx_v], add=True)

        plsc.subcore_barrier()
        pltpu.sync_copy(accum.at[pl.ds(sid * NPS, NPS)],
                        part_hbm.at[cid, pl.ds(sid * NPS, NPS)])

    return k(eo, dst)


def _node_body(*refs):
    part_refs = refs[:-8]
    h_ref, wh_ref, wm_ref, b1_ref, w2_ref, b2_ref, z_ref, h_out_ref = refs[-8:]
    acc = None
    for pr in part_refs:
        pall = pr[...]
        for c in range(NC):
            p = pall[c]
            acc = p if acc is None else acc + p
    m = acc[:, :HID]
    zsum = acc[:, HID:HID + 48]
    cnt = acc[:, HID + 48:HID + 49]
    z_ref[...] = zsum / jnp.maximum(cnt, 1.0)
    pre = (jnp.dot(h_ref[...], wh_ref[...], preferred_element_type=jnp.float32)
           + jnp.dot(m, wm_ref[...], preferred_element_type=jnp.float32)
           + b1_ref[...])
    h_out_ref[...] = (jnp.dot(_silu(pre), w2_ref[...],
                              preferred_element_type=jnp.float32) + b2_ref[...])


def _tc_node(parts, h, wh, wm, b1, w2, b2, *, bn=1000, interpret=False):
    grid = (N // bn,)
    row_spec = lambda w: pl.BlockSpec((bn, w), lambda i: (i, 0))
    part_spec = pl.BlockSpec((NC, bn, ROW), lambda i: (0, i, 0))
    full = lambda a: pl.BlockSpec(a.shape, lambda i: (0,) * a.ndim)
    return pl.pallas_call(
        _node_body,
        grid=grid,
        in_specs=[part_spec] * len(parts)
        + [row_spec(SCALAR_IN), full(wh), full(wm), full(b1), full(w2),
           full(b2)],
        out_specs=[row_spec(48), row_spec(SCALAR_IN)],
        out_shape=[jax.ShapeDtypeStruct((N, 48), jnp.float32),
                   jax.ShapeDtypeStruct((N, SCALAR_IN), jnp.float32)],
        interpret=interpret,
    )(*parts, h, wh, wm, b1, w2, b2)


def kernel(Z, h, edge_index, edge_distance_feature, edge_distance_vec,
           edge_distance, edge_fc_W, edge_fc_b, msg_W1, msg_b1, msg_W2, msg_b2,
           vec_W1, vec_b1, vec_W2, vec_b2, sc_W1, sc_b1, sc_W2, sc_b2):
    src = edge_index[0]
    dst = edge_index[1]
    zf = Z.reshape(N, 48)

    ts, td = _tc_pre(h, zf, msg_W1[0:128], msg_W1[128:256],
                     msg_b1.reshape(1, HID))

    parts = []
    for s in range(NSLICE):
        lo, hi = s * ES, (s + 1) * ES
        src_s, dst_s = src[lo:hi], dst[lo:hi]
        gs, gd = _sc_gather(ts, td, src_s, dst_s)
        eo = _tc_edge(
            gs, gd, edge_distance_feature[lo:hi], edge_distance_vec[lo:hi],
            edge_fc_W, edge_fc_b.reshape(1, HID),
            msg_W1[256:356], msg_W1[356:420],
            msg_W2, msg_b2.reshape(1, HID),
            vec_W1, vec_b1.reshape(1, HID), vec_W2, vec_b2.reshape(1, 80))
        parts.append(_sc_scatter(eo, dst_s))

    z_out, h_out = _tc_node(
        parts, h,
        sc_W1[0:SCALAR_IN], sc_W1[SCALAR_IN:SCALAR_IN + HID],
        sc_b1.reshape(1, HID), sc_W2, sc_b2.reshape(1, SCALAR_OUT))

    return (z_out.reshape(N, 3, VEC_IN), h_out)
